# baseline (speedup 1.0000x reference)
import functools

import jax
import jax.numpy as jnp
from jax import lax
from jax.experimental import pallas as pl
from jax.experimental.pallas import tpu as pltpu
from jax.experimental.pallas import tpu_sc as plsc

VOCAB = 1000000
CUT1 = 20000
CUT2 = 100000
D = 128

N = 4096 * 50
NC, NS = 2, 16
NW = NC * NS
NT = N // NW
C = 1280
NCHUNK = NT // C
G = 128

BT = 3200


def _runs(lo, hi, unit):
    pts = [lo]
    p = (lo // unit + 1) * unit
    while p < hi:
        pts.append(p)
        p += unit
    pts.append(hi)
    return list(zip(pts[:-1], pts[1:]))


_mesh = plsc.VectorSubcoreMesh(core_axis_name="c", subcore_axis_name="s")


@functools.partial(
    pl.kernel,
    out_type=(
        jax.ShapeDtypeStruct((N // 4, 128), jnp.float32),
        jax.ShapeDtypeStruct((N // 16, 128), jnp.float32),
    ),
    mesh=_mesh,
    scratch_types=[
        pltpu.VMEM((C,), jnp.int32),
        pltpu.VMEM((C,), jnp.int32),
        pltpu.VMEM((C + G,), jnp.int32),
        pltpu.VMEM((C + G,), jnp.int32),
        pltpu.VMEM((G, 32), jnp.float32),
        pltpu.VMEM((C, 32), jnp.float32),
        pltpu.VMEM((C, 8), jnp.float32),
        pltpu.SemaphoreType.DMA,
        pltpu.SemaphoreType.DMA,
    ],
    compiler_params=pltpu.CompilerParams(
        use_tc_tiling_on_sc=False, needs_layout_passes=False),
)
def _sc_gather12(ids_hbm, emb1_hbm, emb2_hbm, g1_hbm, g2_hbm,
                 ids_v, rel2_v, rel1c_v, pos1_v, rows1_blk, rows1_v,
                 rows2_v, sem, sem1):
    wid = lax.axis_index("s") * NC + lax.axis_index("c")
    wbase = wid * NT
    iota16 = lax.iota(jnp.int32, 16)
    zeros16 = jnp.zeros((16,), jnp.int32)
    ones16 = jnp.ones((16,), jnp.int32)

    for chunk in range(NCHUNK):
        base = wbase + chunk * C
        lo = chunk * C
        pltpu.sync_copy(ids_hbm.at[pl.ds(base, C)], ids_v)

        copies = []
        for a, b in _runs(lo, lo + C, BT // 4):
            s = (a % BT) // (BT // 4)
            row_base = wid * (2 * (BT // 4)) + (a // BT) * (BT // 4) + a % (BT // 4)
            id0 = ids_v[pl.ds(a - lo, 16)][0]
            s_rel1 = jnp.where(
                jnp.logical_and(id0 >= CUT1, id0 < CUT2),
                id0 - CUT1, id0 & 0xFFFF)

            def prefill(v, _):
                sl = pl.ds((a - lo) + v * 16, 16)
                pos1_v[sl] = zeros16 + row_base
                rel1c_v[sl] = zeros16 + s_rel1
                return 0

            lax.fori_loop(0, (b - a) // 16 + G // 16, prefill, 0)

            def vbody(v, n1):
                sl = pl.ds((a - lo) + v * 16, 16)
                idv = ids_v[sl]
                m1 = jnp.logical_and(idv >= CUT1, idv < CUT2)
                m2 = idv >= CUT2
                rel2_v[sl] = jnp.where(m2, idv - CUT2, idv & 0x7FFFF)
                posv = iota16 + (row_base + v * 16)
                cnt = jnp.where(m1, ones16, zeros16)
                incl = jnp.cumsum(cnt)
                dst = (a - lo) + n1 + (incl - cnt)
                plsc.store_scatter(pos1_v, [dst], posv, mask=m1)
                plsc.store_scatter(rel1c_v, [dst], idv - CUT1, mask=m1)
                return n1 + incl[15]

            n1 = lax.fori_loop(0, (b - a) // 16, vbody, jnp.int32(0))

            for off in range(a - lo, b - lo, G):
                ln = min(G, (b - lo) - off)
                isl = pl.ds(off, ln)
                copies.append(pltpu.async_copy(
                    emb2_hbm.at[rel2_v.at[isl]], rows2_v.at[isl], sem))

            nb1 = (n1 + G - 1) // G

            def b1body(i, _):
                bsl = pl.ds((a - lo) + i * G, G)
                pltpu.async_copy(
                    emb1_hbm.at[rel1c_v.at[bsl]], rows1_blk, sem1).wait()

                def expand(jg, _):
                    posg = pos1_v[pl.ds((a - lo) + i * G + jg * 16, 16)]
                    tok = posg - row_base + (a - lo)
                    eidx = iota16 + jg * 16
                    for k in range(32):
                        kv = zeros16 + k
                        val = plsc.load_gather(rows1_blk, [eidx, kv])
                        plsc.store_scatter(rows1_v, [tok, kv], val)
                    return 0

                lax.fori_loop(0, G // 16, expand, 0)
                return 0

            lax.fori_loop(0, nb1, b1body, 0)

        for cp in copies:
            cp.wait()
        for a, b in _runs(lo, lo + C, BT // 4):
            s = (a % BT) // (BT // 4)
            row = wid * (2 * (BT // 4)) + (a // BT) * (BT // 4) + a % (BT // 4)
            pltpu.sync_copy(
                rows1_v.at[pl.ds(a - lo, b - a), :],
                g1_hbm.at[pl.ds(row, b - a), pl.ds(s * 32, 32)])
        for a, b in _runs(lo, lo + C, BT // 16):
            s2 = (a % BT) // (BT // 16)
            row = wid * (2 * (BT // 16)) + (a // BT) * (BT // 16) + a % (BT // 16)
            pltpu.sync_copy(
                rows2_v.at[pl.ds(a - lo, b - a), :],
                g2_hbm.at[pl.ds(row, b - a), pl.ds(s2 * 8, 8)])


@functools.partial(
    pl.kernel,
    out_type=jax.ShapeDtypeStruct((N, D), jnp.float32),
    mesh=_mesh,
    scratch_types=[
        pltpu.VMEM((C,), jnp.int32),
        pltpu.VMEM((C,), jnp.int32),
        pltpu.VMEM((C,), jnp.int32),
        pltpu.VMEM((G,), jnp.int32),
        pltpu.VMEM((G, D), jnp.float32),
        pltpu.SemaphoreType.DMA,
    ],
    compiler_params=pltpu.CompilerParams(
        use_tc_tiling_on_sc=True, needs_layout_passes=False),
)
def _sc_bucket0(ids_hbm, emb0_hbm, prev_hbm,
                ids_v, rel0_v, pos0_v, pos_blk, rows0_v, sem2):
    wid = lax.axis_index("s") * NC + lax.axis_index("c")
    wbase = wid * NT
    iota16 = lax.iota(jnp.int32, 16)
    zeros16 = jnp.zeros((16,), jnp.int32)
    ones16 = jnp.ones((16,), jnp.int32)

    for chunk in range(NCHUNK):
        base = wbase + chunk * C
        pltpu.sync_copy(ids_hbm.at[pl.ds(base, C)], ids_v)

        s_rel0 = jnp.minimum(ids_v[pl.ds(0, 16)][0], CUT1 - 1)

        def prefill(v, _):
            sl = pl.ds(v * 16, 16)
            pos0_v[sl] = zeros16 + base
            rel0_v[sl] = zeros16 + s_rel0
            return 0

        lax.fori_loop(0, C // 16, prefill, 0)

        def vbody(v, n0):
            sl = pl.ds(v * 16, 16)
            idv = ids_v[sl]
            m0 = idv < CUT1
            posv = iota16 + (base + v * 16)
            cnt = jnp.where(m0, ones16, zeros16)
            incl = jnp.cumsum(cnt)
            dst = n0 + (incl - cnt)
            plsc.store_scatter(pos0_v, [dst], posv, mask=m0)
            plsc.store_scatter(rel0_v, [dst], idv, mask=m0)
            return n0 + incl[15]

        n0 = lax.fori_loop(0, C // 16, vbody, jnp.int32(0))

        nb = (n0 + G - 1) // G

        def b0body(i, _):
            bsl = pl.ds(i * G, G)
            for j in range(G // 16):
                pos_blk[pl.ds(j * 16, 16)] = pos0_v[pl.ds(i * G + j * 16, 16)]
            pltpu.async_copy(emb0_hbm.at[rel0_v.at[bsl]], rows0_v, sem2).wait()
            pltpu.async_copy(rows0_v, prev_hbm.at[pos_blk], sem2).wait()
            return 0

        lax.fori_loop(0, nb, b0body, 0)


def _tc_body(ids_ref, g1_ref, g2_ref, prev_ref, w1_ref, w2_ref, out_ref):
    ids = ids_ref[...]
    prev = prev_ref[...]
    e1big = lax.dot_general(g1_ref[...], w1_ref[...],
                            (((1,), (0,)), ((), ())),
                            preferred_element_type=jnp.float32)
    e2big = lax.dot_general(g2_ref[...], w2_ref[...],
                            (((1,), (0,)), ((), ())),
                            preferred_element_type=jnp.float32)
    R2 = BT // 16
    for s2 in range(16):
        s, q = s2 // 4, s2 % 4
        e1p = e1big[q * R2:(q + 1) * R2, s * D:(s + 1) * D]
        e2p = e2big[:, s2 * D:(s2 + 1) * D]
        idp = ids[s2 * R2:(s2 + 1) * R2, :]
        sel = jnp.where(idp >= CUT2, e2p, e1p)
        out_ref[s2 * R2:(s2 + 1) * R2, :] = jnp.where(
            idp < CUT1, prev[s2 * R2:(s2 + 1) * R2, :], sel)


@jax.jit
def kernel(input_ids, emb0, emb1, emb2, proj1, proj2):
    B, L = input_ids.shape
    assert B * L == N
    ids = input_ids.reshape(-1).astype(jnp.int32)

    g1, g2 = _sc_gather12(ids, emb1, emb2)
    prev = _sc_bucket0(ids, emb0)

    eye4 = jnp.eye(4, dtype=jnp.float32)
    eye16 = jnp.eye(16, dtype=jnp.float32)
    w1big = jnp.kron(eye4, proj1.T)
    w2big = jnp.kron(eye16, proj2.T)

    out = pl.pallas_call(
        _tc_body,
        grid=(N // BT,),
        in_specs=[
            pl.BlockSpec((BT, 1), lambda i: (i, 0)),
            pl.BlockSpec((BT // 4, 128), lambda i: (i, 0)),
            pl.BlockSpec((BT // 16, 128), lambda i: (i, 0)),
            pl.BlockSpec((BT, D), lambda i: (i, 0)),
            pl.BlockSpec((128, 512), lambda i: (0, 0)),
            pl.BlockSpec((128, 2048), lambda i: (0, 0)),
        ],
        out_specs=pl.BlockSpec((BT, D), lambda i: (i, 0)),
        out_shape=jax.ShapeDtypeStruct((N, D), jnp.float32),
    )(ids.reshape(N, 1), g1, g2, prev, w1big, w2big)

    return out.reshape(B, L, D)


# speedup vs baseline: 1.0609x; 1.0609x over previous
import functools

import jax
import jax.numpy as jnp
from jax import lax
from jax.experimental import pallas as pl
from jax.experimental.pallas import tpu as pltpu
from jax.experimental.pallas import tpu_sc as plsc

VOCAB = 1000000
CUT1 = 20000
CUT2 = 100000
D = 128

N = 4096 * 50
NC, NS = 2, 16
NW = NC * NS
NT = N // NW
C = 1280
NCHUNK = NT // C
G = 128

BT = 3200


def _runs(lo, hi, unit):
    pts = [lo]
    p = (lo // unit + 1) * unit
    while p < hi:
        pts.append(p)
        p += unit
    pts.append(hi)
    return list(zip(pts[:-1], pts[1:]))


_mesh = plsc.VectorSubcoreMesh(core_axis_name="c", subcore_axis_name="s")


@functools.partial(
    pl.kernel,
    out_type=(
        jax.ShapeDtypeStruct((N // 4, 128), jnp.float32),
        jax.ShapeDtypeStruct((N // 16, 128), jnp.float32),
    ),
    mesh=_mesh,
    scratch_types=[
        pltpu.VMEM((2, C), jnp.int32),
        pltpu.VMEM((2, C), jnp.int32),
        pltpu.VMEM((2, C), jnp.int32),
        pltpu.VMEM((2, C, 32), jnp.float32),
        pltpu.VMEM((2, C, 8), jnp.float32),
        pltpu.SemaphoreType.DMA,
        pltpu.SemaphoreType.DMA,
    ],
    compiler_params=pltpu.CompilerParams(
        use_tc_tiling_on_sc=False, needs_layout_passes=False),
)
def _sc_gather12(ids_hbm, emb1_hbm, emb2_hbm, g1_hbm, g2_hbm,
                 ids_v, rel1_v, rel2_v, rows1_v, rows2_v, sem_a, sem_b):
    wid = lax.axis_index("s") * NC + lax.axis_index("c")
    wbase = wid * NT
    sems = (sem_a, sem_b)

    def fire(chunk):
        p = chunk % 2
        base = wbase + chunk * C
        idsp, rel1p, rel2p = ids_v.at[p], rel1_v.at[p], rel2_v.at[p]
        pltpu.sync_copy(ids_hbm.at[pl.ds(base, C)], idsp)

        def vbody(v, carry):
            sl = pl.ds(v * 16, 16)
            idv = idsp[sl]
            m1 = jnp.logical_and(idv >= CUT1, idv < CUT2)
            m2 = idv >= CUT2
            rel1p[sl] = jnp.where(m1, idv - CUT1, idv & 0xFFFF)
            rel2p[sl] = jnp.where(m2, idv - CUT2, idv & 0x7FFFF)
            return carry

        lax.fori_loop(0, C // 16, vbody, jnp.int32(0))

        copies = []
        for i in range(C // G):
            isl = pl.ds(i * G, G)
            copies.append(pltpu.async_copy(
                emb1_hbm.at[rel1p.at[isl]], rows1_v.at[p].at[isl], sems[p]))
            copies.append(pltpu.async_copy(
                emb2_hbm.at[rel2p.at[isl]], rows2_v.at[p].at[isl], sems[p]))
        return copies

    def drain_and_write(chunk, copies):
        p = chunk % 2
        lo = chunk * C
        for cp in copies:
            cp.wait()
        for a, b in _runs(lo, lo + C, BT // 4):
            s = (a % BT) // (BT // 4)
            row = wid * (2 * (BT // 4)) + (a // BT) * (BT // 4) + a % (BT // 4)
            pltpu.sync_copy(
                rows1_v.at[p].at[pl.ds(a - lo, b - a), :],
                g1_hbm.at[pl.ds(row, b - a), pl.ds(s * 32, 32)])
        for a, b in _runs(lo, lo + C, BT // 16):
            s2 = (a % BT) // (BT // 16)
            row = wid * (2 * (BT // 16)) + (a // BT) * (BT // 16) + a % (BT // 16)
            pltpu.sync_copy(
                rows2_v.at[p].at[pl.ds(a - lo, b - a), :],
                g2_hbm.at[pl.ds(row, b - a), pl.ds(s2 * 8, 8)])

    pending = fire(0)
    for chunk in range(NCHUNK):
        nxt = fire(chunk + 1) if chunk + 1 < NCHUNK else None
        drain_and_write(chunk, pending)
        pending = nxt


@functools.partial(
    pl.kernel,
    out_type=jax.ShapeDtypeStruct((N, D), jnp.float32),
    mesh=_mesh,
    scratch_types=[
        pltpu.VMEM((C,), jnp.int32),
        pltpu.VMEM((C,), jnp.int32),
        pltpu.VMEM((C,), jnp.int32),
        pltpu.VMEM((G,), jnp.int32),
        pltpu.VMEM((G, D), jnp.float32),
        pltpu.SemaphoreType.DMA,
    ],
    compiler_params=pltpu.CompilerParams(
        use_tc_tiling_on_sc=True, needs_layout_passes=False),
)
def _sc_bucket0(ids_hbm, emb0_hbm, prev_hbm,
                ids_v, rel0_v, pos0_v, pos_blk, rows0_v, sem2):
    wid = lax.axis_index("s") * NC + lax.axis_index("c")
    wbase = wid * NT
    iota16 = lax.iota(jnp.int32, 16)
    zeros16 = jnp.zeros((16,), jnp.int32)
    ones16 = jnp.ones((16,), jnp.int32)

    for chunk in range(NCHUNK):
        base = wbase + chunk * C
        pltpu.sync_copy(ids_hbm.at[pl.ds(base, C)], ids_v)

        s_rel0 = jnp.minimum(ids_v[pl.ds(0, 16)][0], CUT1 - 1)

        def prefill(v, _):
            sl = pl.ds(v * 16, 16)
            pos0_v[sl] = zeros16 + base
            rel0_v[sl] = zeros16 + s_rel0
            return 0

        lax.fori_loop(0, C // 16, prefill, 0)

        def vbody(v, n0):
            sl = pl.ds(v * 16, 16)
            idv = ids_v[sl]
            m0 = idv < CUT1
            posv = iota16 + (base + v * 16)
            cnt = jnp.where(m0, ones16, zeros16)
            incl = jnp.cumsum(cnt)
            dst = n0 + (incl - cnt)
            plsc.store_scatter(pos0_v, [dst], posv, mask=m0)
            plsc.store_scatter(rel0_v, [dst], idv, mask=m0)
            return n0 + incl[15]

        n0 = lax.fori_loop(0, C // 16, vbody, jnp.int32(0))

        nb = (n0 + G - 1) // G

        def b0body(i, _):
            bsl = pl.ds(i * G, G)
            for j in range(G // 16):
                pos_blk[pl.ds(j * 16, 16)] = pos0_v[pl.ds(i * G + j * 16, 16)]
            pltpu.async_copy(emb0_hbm.at[rel0_v.at[bsl]], rows0_v, sem2).wait()
            pltpu.async_copy(rows0_v, prev_hbm.at[pos_blk], sem2).wait()
            return 0

        lax.fori_loop(0, nb, b0body, 0)


def _tc_body(ids_ref, g1_ref, g2_ref, prev_ref, w1_ref, w2_ref, out_ref):
    ids = ids_ref[...]
    prev = prev_ref[...]
    e1big = lax.dot_general(g1_ref[...], w1_ref[...],
                            (((1,), (0,)), ((), ())),
                            preferred_element_type=jnp.float32)
    e2big = lax.dot_general(g2_ref[...], w2_ref[...],
                            (((1,), (0,)), ((), ())),
                            preferred_element_type=jnp.float32)
    R2 = BT // 16
    for s2 in range(16):
        s, q = s2 // 4, s2 % 4
        e1p = e1big[q * R2:(q + 1) * R2, s * D:(s + 1) * D]
        e2p = e2big[:, s2 * D:(s2 + 1) * D]
        idp = ids[s2 * R2:(s2 + 1) * R2, :]
        sel = jnp.where(idp >= CUT2, e2p, e1p)
        out_ref[s2 * R2:(s2 + 1) * R2, :] = jnp.where(
            idp < CUT1, prev[s2 * R2:(s2 + 1) * R2, :], sel)


@jax.jit
def kernel(input_ids, emb0, emb1, emb2, proj1, proj2):
    B, L = input_ids.shape
    assert B * L == N
    ids = input_ids.reshape(-1).astype(jnp.int32)

    g1, g2 = _sc_gather12(ids, emb1, emb2)
    prev = _sc_bucket0(ids, emb0)

    eye4 = jnp.eye(4, dtype=jnp.float32)
    eye16 = jnp.eye(16, dtype=jnp.float32)
    w1big = jnp.kron(eye4, proj1.T)
    w2big = jnp.kron(eye16, proj2.T)

    out = pl.pallas_call(
        _tc_body,
        grid=(N // BT,),
        in_specs=[
            pl.BlockSpec((BT, 1), lambda i: (i, 0)),
            pl.BlockSpec((BT // 4, 128), lambda i: (i, 0)),
            pl.BlockSpec((BT // 16, 128), lambda i: (i, 0)),
            pl.BlockSpec((BT, D), lambda i: (i, 0)),
            pl.BlockSpec((128, 512), lambda i: (0, 0)),
            pl.BlockSpec((128, 2048), lambda i: (0, 0)),
        ],
        out_specs=pl.BlockSpec((BT, D), lambda i: (i, 0)),
        out_shape=jax.ShapeDtypeStruct((N, D), jnp.float32),
    )(ids.reshape(N, 1), g1, g2, prev, w1big, w2big)

    return out.reshape(B, L, D)
